# baseline (speedup 1.0000x reference)
import jax
import jax.numpy as jnp
from jax.experimental import pallas as pl
from jax.experimental.pallas import tpu as pltpu

H = 2048
M = 1408
MS = 5632
E = 16
T = 32

MC = 128
NJ = M // MC
MSC = 512
NSJ = MS // MSC


def _router_body(flat_ref, rw_ref, logits_ref, routing_ref):
    flat = flat_ref[...]
    logits = jnp.dot(flat, rw_ref[...], preferred_element_type=jnp.float32)
    logits_ref[...] = logits
    m = jnp.max(logits, axis=-1, keepdims=True)
    ex = jnp.exp(logits - m)
    probs = ex / jnp.sum(ex, axis=-1, keepdims=True)
    lane = jax.lax.broadcasted_iota(jnp.int32, probs.shape, 1)
    p1 = jnp.max(probs, axis=-1, keepdims=True)
    i1 = jnp.min(jnp.where(probs == p1, lane, E), axis=-1, keepdims=True)
    is1 = lane == i1
    probs2 = jnp.where(is1, -1.0, probs)
    p2 = jnp.max(probs2, axis=-1, keepdims=True)
    i2 = jnp.min(jnp.where(probs2 == p2, lane, E), axis=-1, keepdims=True)
    is2 = lane == i2
    s = p1 + p2
    routing_ref[...] = jnp.where(is1, p1 / s, 0.0) + jnp.where(is2, p2 / s, 0.0)


def _expert_body(flat_ref, routing_ref, gate_ref, up_ref, outw_ref, acc_ref):
    e = pl.program_id(0)
    j = pl.program_id(1)
    flat = flat_ref[...]
    g = jnp.dot(flat, gate_ref[0], preferred_element_type=jnp.float32)
    u = jnp.dot(flat, up_ref[0], preferred_element_type=jnp.float32)
    lane = jax.lax.broadcasted_iota(jnp.int32, (T, E), 1)
    w = jnp.sum(jnp.where(lane == e, routing_ref[...], 0.0), axis=1,
                keepdims=True)
    h = (g * jax.nn.sigmoid(g)) * u * w
    contrib = jnp.dot(h, outw_ref[0], preferred_element_type=jnp.float32)

    @pl.when(jnp.logical_and(e == 0, j == 0))
    def _init():
        acc_ref[...] = jnp.zeros_like(acc_ref)

    acc_ref[...] += contrib


def _shared_body(flat_ref, base_ref, eg_ref, gate_ref, inter_ref, outw_ref,
                 out_ref):
    j = pl.program_id(0)
    flat = flat_ref[...]
    seg = jax.nn.sigmoid(
        jnp.dot(flat, eg_ref[...], preferred_element_type=jnp.float32))
    g = jnp.dot(flat, gate_ref[...], preferred_element_type=jnp.float32)
    x = jnp.dot(flat, inter_ref[...], preferred_element_type=jnp.float32)
    h = x * (g * jax.nn.sigmoid(g)) * seg
    contrib = jnp.dot(h, outw_ref[...], preferred_element_type=jnp.float32)

    @pl.when(j == 0)
    def _init():
        out_ref[...] = base_ref[...]

    out_ref[...] += contrib


def kernel(hidden_states, router_w, expert_gate_w, expert_out_w,
           shared_gate_w, shared_inter_w, shared_out_w, shared_eg_w):
    B, S, _ = hidden_states.shape
    flat = hidden_states.reshape(-1, H)

    logits, routing = pl.pallas_call(
        _router_body,
        out_shape=(
            jax.ShapeDtypeStruct((T, E), jnp.float32),
            jax.ShapeDtypeStruct((T, E), jnp.float32),
        ),
    )(flat, router_w)

    expert_acc = pl.pallas_call(
        _expert_body,
        grid=(E, NJ),
        in_specs=[
            pl.BlockSpec((T, H), lambda e, j: (0, 0)),
            pl.BlockSpec((T, E), lambda e, j: (0, 0)),
            pl.BlockSpec((1, H, MC), lambda e, j: (e, 0, j)),
            pl.BlockSpec((1, H, MC), lambda e, j: (e, 0, j + NJ)),
            pl.BlockSpec((1, MC, H), lambda e, j: (e, j, 0)),
        ],
        out_specs=pl.BlockSpec((T, H), lambda e, j: (0, 0)),
        out_shape=jax.ShapeDtypeStruct((T, H), jnp.float32),
        compiler_params=pltpu.CompilerParams(
            dimension_semantics=("arbitrary", "arbitrary")),
    )(flat, routing, expert_gate_w, expert_gate_w, expert_out_w)

    out_flat = pl.pallas_call(
        _shared_body,
        grid=(NSJ,),
        in_specs=[
            pl.BlockSpec((T, H), lambda j: (0, 0)),
            pl.BlockSpec((T, H), lambda j: (0, 0)),
            pl.BlockSpec((H, 1), lambda j: (0, 0)),
            pl.BlockSpec((H, MSC), lambda j: (0, j)),
            pl.BlockSpec((H, MSC), lambda j: (0, j)),
            pl.BlockSpec((MSC, H), lambda j: (j, 0)),
        ],
        out_specs=pl.BlockSpec((T, H), lambda j: (0, 0)),
        out_shape=jax.ShapeDtypeStruct((T, H), jnp.float32),
        compiler_params=pltpu.CompilerParams(
            dimension_semantics=("arbitrary",)),
    )(flat, expert_acc, shared_eg_w, shared_gate_w, shared_inter_w,
      shared_out_w)

    return (out_flat.reshape(B, S, H), logits)


# speedup vs baseline: 1.2408x; 1.2408x over previous
import jax
import jax.numpy as jnp
from jax.experimental import pallas as pl
from jax.experimental.pallas import tpu as pltpu

H = 2048
M = 1408
MS = 5632
E = 16
T = 32

HC = 512
NH = H // HC
SHC = 256
NSH = H // SHC
SKC = 512
NSK = MS // SKC


def _router_body(flat_ref, rw_ref, logits_ref, routing_ref):
    flat = flat_ref[...]
    logits = jnp.dot(flat, rw_ref[...], preferred_element_type=jnp.float32)
    logits_ref[...] = logits
    m = jnp.max(logits, axis=-1, keepdims=True)
    ex = jnp.exp(logits - m)
    probs = ex / jnp.sum(ex, axis=-1, keepdims=True)
    lane = jax.lax.broadcasted_iota(jnp.int32, probs.shape, 1)
    p1 = jnp.max(probs, axis=-1, keepdims=True)
    i1 = jnp.min(jnp.where(probs == p1, lane, E), axis=-1, keepdims=True)
    is1 = lane == i1
    probs2 = jnp.where(is1, -1.0, probs)
    p2 = jnp.max(probs2, axis=-1, keepdims=True)
    i2 = jnp.min(jnp.where(probs2 == p2, lane, E), axis=-1, keepdims=True)
    is2 = lane == i2
    s = p1 + p2
    routing_ref[...] = jnp.where(is1, p1 / s, 0.0) + jnp.where(is2, p2 / s, 0.0)


def _expert_body(flat_ref, routing_ref, gate_ref, outw_ref, acc_ref, gu_ref):
    e = pl.program_id(0)
    jh = pl.program_id(1)
    part = jnp.dot(flat_ref[...], gate_ref[0],
                   preferred_element_type=jnp.float32)

    @pl.when(jh == 0)
    def _reset():
        gu_ref[...] = part

    @pl.when(jh != 0)
    def _accum():
        gu_ref[...] += part

    @pl.when(jh == NH - 1)
    def _down():
        gu = gu_ref[...]
        g = gu[:, :M]
        u = gu[:, M:]
        lane = jax.lax.broadcasted_iota(jnp.int32, (T, E), 1)
        w = jnp.sum(jnp.where(lane == e, routing_ref[...], 0.0), axis=1,
                    keepdims=True)
        h = (g * jax.nn.sigmoid(g)) * u * w
        contrib = jnp.dot(h, outw_ref[0], preferred_element_type=jnp.float32)

        @pl.when(e == 0)
        def _init():
            acc_ref[...] = contrib

        @pl.when(e != 0)
        def _add():
            acc_ref[...] += contrib


def _shared_body(flat_ref, base_ref, eg_ref, gate_ref, inter_ref, outw_ref,
                 out_ref, g_ref, x_ref, h_ref, seg_ref):
    j = pl.program_id(0)

    @pl.when(j < NSH)
    def _phase1():
        fc = flat_ref[...]
        gp = jnp.dot(fc, gate_ref[...], preferred_element_type=jnp.float32)
        xp = jnp.dot(fc, inter_ref[...], preferred_element_type=jnp.float32)
        sp = jnp.dot(fc, eg_ref[...], preferred_element_type=jnp.float32)

        @pl.when(j == 0)
        def _reset():
            g_ref[...] = gp
            x_ref[...] = xp
            seg_ref[...] = sp

        @pl.when(j != 0)
        def _accum():
            g_ref[...] += gp
            x_ref[...] += xp
            seg_ref[...] += sp

    @pl.when(j == NSH)
    def _activate():
        g = g_ref[...]
        seg = jax.nn.sigmoid(seg_ref[...])
        h_ref[...] = x_ref[...] * (g * jax.nn.sigmoid(g)) * seg
        out_ref[...] = base_ref[...]

    @pl.when(j > NSH)
    def _phase2():
        k = j - NSH - 1
        hc = h_ref[:, pl.ds(k * SKC, SKC)]
        out_ref[...] += jnp.dot(hc, outw_ref[...],
                                preferred_element_type=jnp.float32)


def kernel(hidden_states, router_w, expert_gate_w, expert_out_w,
           shared_gate_w, shared_inter_w, shared_out_w, shared_eg_w):
    B, S, _ = hidden_states.shape
    flat = hidden_states.reshape(-1, H)

    logits, routing = pl.pallas_call(
        _router_body,
        out_shape=(
            jax.ShapeDtypeStruct((T, E), jnp.float32),
            jax.ShapeDtypeStruct((T, E), jnp.float32),
        ),
    )(flat, router_w)

    expert_acc = pl.pallas_call(
        _expert_body,
        grid=(E, NH),
        in_specs=[
            pl.BlockSpec((T, HC), lambda e, j: (0, j)),
            pl.BlockSpec((T, E), lambda e, j: (0, 0)),
            pl.BlockSpec((1, HC, 2 * M), lambda e, j: (e, j, 0)),
            pl.BlockSpec((1, M, H), lambda e, j: (e, 0, 0)),
        ],
        out_specs=pl.BlockSpec((T, H), lambda e, j: (0, 0)),
        out_shape=jax.ShapeDtypeStruct((T, H), jnp.float32),
        scratch_shapes=[pltpu.VMEM((T, 2 * M), jnp.float32)],
        compiler_params=pltpu.CompilerParams(
            dimension_semantics=("arbitrary", "arbitrary")),
    )(flat, routing, expert_gate_w, expert_out_w)

    nsteps = NSH + 1 + NSK
    out_flat = pl.pallas_call(
        _shared_body,
        grid=(nsteps,),
        in_specs=[
            pl.BlockSpec((T, SHC), lambda j: (0, jnp.minimum(j, NSH - 1))),
            pl.BlockSpec((T, H), lambda j: (0, 0)),
            pl.BlockSpec((SHC, 1), lambda j: (jnp.minimum(j, NSH - 1), 0)),
            pl.BlockSpec((SHC, MS), lambda j: (jnp.minimum(j, NSH - 1), 0)),
            pl.BlockSpec((SHC, MS), lambda j: (jnp.minimum(j, NSH - 1), 0)),
            pl.BlockSpec((SKC, H),
                         lambda j: (jnp.clip(j - NSH - 1, 0, NSK - 1), 0)),
        ],
        out_specs=pl.BlockSpec((T, H), lambda j: (0, 0)),
        out_shape=jax.ShapeDtypeStruct((T, H), jnp.float32),
        scratch_shapes=[
            pltpu.VMEM((T, MS), jnp.float32),
            pltpu.VMEM((T, MS), jnp.float32),
            pltpu.VMEM((T, MS), jnp.float32),
            pltpu.VMEM((T, 1), jnp.float32),
        ],
        compiler_params=pltpu.CompilerParams(
            dimension_semantics=("arbitrary",)),
    )(flat, expert_acc, shared_eg_w, shared_gate_w, shared_inter_w,
      shared_out_w)

    return (out_flat.reshape(B, S, H), logits)


# speedup vs baseline: 1.2728x; 1.0258x over previous
import jax
import jax.numpy as jnp
from jax.experimental import pallas as pl
from jax.experimental.pallas import tpu as pltpu

H = 2048
M = 1408
MS = 5632
E = 16
T = 32

P1C = 128
NP1 = H // P1C
EHC = 512
NEH = H // EHC
DC = M // NEH
SKC = 512
NSK = MS // SKC

J_ACT = NP1
J_E0 = J_ACT + 1
J_D0 = J_E0 + NEH
J_P2 = J_E0 + E * NEH + NEH
NSTEPS = J_P2 + NSK


def _body(flat_ref, rw_ref, eg_ref, sg_ref, si_ref, gate_ref, outw_ref,
          so_ref, out_ref, logits_ref, g_ref, x_ref, h_sh_ref, seg_ref,
          rout_ref, gu_ref, he_ref):
    j = pl.program_id(0)

    @pl.when(j == 0)
    def _router():
        flat = flat_ref[...]
        logits = jnp.dot(flat, rw_ref[...], preferred_element_type=jnp.float32)
        logits_ref[...] = logits
        m = jnp.max(logits, axis=-1, keepdims=True)
        ex = jnp.exp(logits - m)
        probs = ex / jnp.sum(ex, axis=-1, keepdims=True)
        lane = jax.lax.broadcasted_iota(jnp.int32, probs.shape, 1)
        p1 = jnp.max(probs, axis=-1, keepdims=True)
        i1 = jnp.min(jnp.where(probs == p1, lane, E), axis=-1, keepdims=True)
        is1 = lane == i1
        probs2 = jnp.where(is1, -1.0, probs)
        p2 = jnp.max(probs2, axis=-1, keepdims=True)
        i2 = jnp.min(jnp.where(probs2 == p2, lane, E), axis=-1, keepdims=True)
        is2 = lane == i2
        s = p1 + p2
        rout_ref[...] = (jnp.where(is1, p1 / s, 0.0)
                         + jnp.where(is2, p2 / s, 0.0))
        seg_ref[...] = jnp.dot(flat, eg_ref[...],
                               preferred_element_type=jnp.float32)

    @pl.when(j < NP1)
    def _phase1():
        fc = flat_ref[:, pl.ds(j * P1C, P1C)]
        gp = jnp.dot(fc, sg_ref[...], preferred_element_type=jnp.float32)
        xp = jnp.dot(fc, si_ref[...], preferred_element_type=jnp.float32)

        @pl.when(j == 0)
        def _reset():
            g_ref[...] = gp
            x_ref[...] = xp

        @pl.when(j != 0)
        def _accum():
            g_ref[...] += gp
            x_ref[...] += xp

    @pl.when(j == J_ACT)
    def _activate():
        g = g_ref[...]
        h_sh_ref[...] = x_ref[...] * (g * jax.nn.sigmoid(g)) * \
            jax.nn.sigmoid(seg_ref[...])
        out_ref[...] = jnp.zeros_like(out_ref)

    @pl.when(jnp.logical_and(j >= J_D0, j < J_P2))
    def _down():
        kd = j - J_D0
        cd = kd % NEH
        out_ref[...] += jnp.dot(he_ref[cd], outw_ref[0],
                                preferred_element_type=jnp.float32)

    @pl.when(jnp.logical_and(j >= J_E0, j < J_E0 + E * NEH))
    def _expert():
        ke = j - J_E0
        k = ke % NEH
        fc = flat_ref[:, pl.ds(k * EHC, EHC)]
        part = jnp.dot(fc, gate_ref[0], preferred_element_type=jnp.float32)

        @pl.when(k == 0)
        def _reset():
            gu_ref[...] = part

        @pl.when(k != 0)
        def _accum():
            gu_ref[...] += part

        @pl.when(k == NEH - 1)
        def _act_e():
            e = ke // NEH
            gu = gu_ref[...]
            g = gu[:, :M]
            u = gu[:, M:]
            lane = jax.lax.broadcasted_iota(jnp.int32, (T, E), 1)
            w = jnp.sum(jnp.where(lane == e, rout_ref[...], 0.0), axis=1,
                        keepdims=True)
            h = (g * jax.nn.sigmoid(g)) * u * w
            for c in range(NEH):
                he_ref[c] = h[:, c * DC:(c + 1) * DC]

    @pl.when(j >= J_P2)
    def _phase2():
        ks = j - J_P2
        hc = h_sh_ref[:, pl.ds(ks * SKC, SKC)]
        out_ref[...] += jnp.dot(hc, so_ref[...],
                                preferred_element_type=jnp.float32)


def kernel(hidden_states, router_w, expert_gate_w, expert_out_w,
           shared_gate_w, shared_inter_w, shared_out_w, shared_eg_w):
    B, S, _ = hidden_states.shape
    flat = hidden_states.reshape(-1, H)

    def _e_idx(j):
        ke = jnp.clip(j - J_E0, 0, E * NEH - 1)
        return (ke // NEH, ke % NEH, 0)

    def _d_idx(j):
        kd = jnp.clip(j - J_D0, 0, E * NEH - 1)
        return (kd // NEH, kd % NEH, 0)

    out_flat, logits = pl.pallas_call(
        _body,
        grid=(NSTEPS,),
        in_specs=[
            pl.BlockSpec((T, H), lambda j: (0, 0)),
            pl.BlockSpec((H, E), lambda j: (0, 0)),
            pl.BlockSpec((H, 1), lambda j: (0, 0)),
            pl.BlockSpec((P1C, MS), lambda j: (jnp.clip(j, 0, NP1 - 1), 0)),
            pl.BlockSpec((P1C, MS), lambda j: (jnp.clip(j, 0, NP1 - 1), 0)),
            pl.BlockSpec((1, EHC, 2 * M), _e_idx),
            pl.BlockSpec((1, DC, H), _d_idx),
            pl.BlockSpec((SKC, H), lambda j: (jnp.clip(j - J_P2, 0, NSK - 1), 0)),
        ],
        out_specs=(
            pl.BlockSpec((T, H), lambda j: (0, 0)),
            pl.BlockSpec((T, E), lambda j: (0, 0)),
        ),
        out_shape=(
            jax.ShapeDtypeStruct((T, H), jnp.float32),
            jax.ShapeDtypeStruct((T, E), jnp.float32),
        ),
        scratch_shapes=[
            pltpu.VMEM((T, MS), jnp.float32),
            pltpu.VMEM((T, MS), jnp.float32),
            pltpu.VMEM((T, MS), jnp.float32),
            pltpu.VMEM((T, 1), jnp.float32),
            pltpu.VMEM((T, E), jnp.float32),
            pltpu.VMEM((T, 2 * M), jnp.float32),
            pltpu.VMEM((NEH, T, DC), jnp.float32),
        ],
        compiler_params=pltpu.CompilerParams(
            dimension_semantics=("arbitrary",)),
    )(flat, router_w, shared_eg_w, shared_gate_w, shared_inter_w,
      expert_gate_w, expert_out_w, shared_out_w)

    return (out_flat.reshape(B, S, H), logits)


# speedup vs baseline: 1.2764x; 1.0028x over previous
import jax
import jax.numpy as jnp
from jax.experimental import pallas as pl
from jax.experimental.pallas import tpu as pltpu

H = 2048
M = 1408
MS = 5632
E = 16
T = 32

P1C = 128
NP1 = H // P1C
EHC = 512
NEH = H // EHC
DCH = H // NEH
SKC = 512
NSK = MS // SKC

J_ACT = NP1
J_E0 = J_ACT + 1
J_D0 = J_E0 + NEH
J_P2 = J_E0 + E * NEH + NEH
NSTEPS = J_P2 + NSK


def _body(flat_ref, rw_ref, eg_ref, sg_ref, si_ref, gate_ref, outw_ref,
          so_ref, out_ref, logits_ref, g_ref, x_ref, h_sh_ref, seg_ref,
          rout_ref, gu_ref, he_ref):
    j = pl.program_id(0)

    @pl.when(j == 0)
    def _router():
        flat = flat_ref[...]
        logits = jnp.dot(flat, rw_ref[...], preferred_element_type=jnp.float32)
        logits_ref[...] = logits
        m = jnp.max(logits, axis=-1, keepdims=True)
        ex = jnp.exp(logits - m)
        probs = ex / jnp.sum(ex, axis=-1, keepdims=True)
        lane = jax.lax.broadcasted_iota(jnp.int32, probs.shape, 1)
        p1 = jnp.max(probs, axis=-1, keepdims=True)
        i1 = jnp.min(jnp.where(probs == p1, lane, E), axis=-1, keepdims=True)
        is1 = lane == i1
        probs2 = jnp.where(is1, -1.0, probs)
        p2 = jnp.max(probs2, axis=-1, keepdims=True)
        i2 = jnp.min(jnp.where(probs2 == p2, lane, E), axis=-1, keepdims=True)
        is2 = lane == i2
        s = p1 + p2
        rout_ref[...] = (jnp.where(is1, p1 / s, 0.0)
                         + jnp.where(is2, p2 / s, 0.0))
        seg_ref[...] = jnp.dot(flat, eg_ref[...],
                               preferred_element_type=jnp.float32)

    @pl.when(j < NP1)
    def _phase1():
        fc = flat_ref[:, pl.ds(j * P1C, P1C)]
        gp = jnp.dot(fc, sg_ref[...], preferred_element_type=jnp.float32)
        xp = jnp.dot(fc, si_ref[...], preferred_element_type=jnp.float32)

        @pl.when(j == 0)
        def _reset():
            g_ref[...] = gp
            x_ref[...] = xp

        @pl.when(j != 0)
        def _accum():
            g_ref[...] += gp
            x_ref[...] += xp

    @pl.when(j == J_ACT)
    def _activate():
        g = g_ref[...]
        h_sh_ref[...] = x_ref[...] * (g * jax.nn.sigmoid(g)) * \
            jax.nn.sigmoid(seg_ref[...])
        out_ref[...] = jnp.zeros_like(out_ref)

    @pl.when(jnp.logical_and(j >= J_D0, j < J_P2))
    def _down():
        kd = j - J_D0
        cd = kd % NEH
        out_ref[:, pl.ds(cd * DCH, DCH)] += jnp.dot(
            he_ref[...], outw_ref[0], preferred_element_type=jnp.float32)

    @pl.when(jnp.logical_and(j >= J_E0, j < J_E0 + E * NEH))
    def _expert():
        ke = j - J_E0
        k = ke % NEH
        fc = flat_ref[:, pl.ds(k * EHC, EHC)]
        part = jnp.dot(fc, gate_ref[0], preferred_element_type=jnp.float32)

        @pl.when(k == 0)
        def _reset():
            gu_ref[...] = part

        @pl.when(k != 0)
        def _accum():
            gu_ref[...] += part

        @pl.when(k == NEH - 1)
        def _act_e():
            e = ke // NEH
            gu = gu_ref[...]
            g = gu[:, :M]
            u = gu[:, M:]
            lane = jax.lax.broadcasted_iota(jnp.int32, (T, E), 1)
            w = jnp.sum(jnp.where(lane == e, rout_ref[...], 0.0), axis=1,
                        keepdims=True)
            he_ref[...] = (g * jax.nn.sigmoid(g)) * u * w

    @pl.when(j >= J_P2)
    def _phase2():
        ks = j - J_P2
        hc = h_sh_ref[:, pl.ds(ks * SKC, SKC)]
        out_ref[...] += jnp.dot(hc, so_ref[...],
                                preferred_element_type=jnp.float32)


def kernel(hidden_states, router_w, expert_gate_w, expert_out_w,
           shared_gate_w, shared_inter_w, shared_out_w, shared_eg_w):
    B, S, _ = hidden_states.shape
    flat = hidden_states.reshape(-1, H)

    def _e_idx(j):
        ke = jnp.clip(j - J_E0, 0, E * NEH - 1)
        return (ke // NEH, ke % NEH, 0)

    def _d_idx(j):
        kd = jnp.clip(j - J_D0, 0, E * NEH - 1)
        return (kd // NEH, 0, kd % NEH)

    out_flat, logits = pl.pallas_call(
        _body,
        grid=(NSTEPS,),
        in_specs=[
            pl.BlockSpec((T, H), lambda j: (0, 0)),
            pl.BlockSpec((H, E), lambda j: (0, 0)),
            pl.BlockSpec((H, 1), lambda j: (0, 0)),
            pl.BlockSpec((P1C, MS), lambda j: (jnp.clip(j, 0, NP1 - 1), 0)),
            pl.BlockSpec((P1C, MS), lambda j: (jnp.clip(j, 0, NP1 - 1), 0)),
            pl.BlockSpec((1, EHC, 2 * M), _e_idx),
            pl.BlockSpec((1, M, DCH), _d_idx),
            pl.BlockSpec((SKC, H), lambda j: (jnp.clip(j - J_P2, 0, NSK - 1), 0)),
        ],
        out_specs=(
            pl.BlockSpec((T, H), lambda j: (0, 0)),
            pl.BlockSpec((T, E), lambda j: (0, 0)),
        ),
        out_shape=(
            jax.ShapeDtypeStruct((T, H), jnp.float32),
            jax.ShapeDtypeStruct((T, E), jnp.float32),
        ),
        scratch_shapes=[
            pltpu.VMEM((T, MS), jnp.float32),
            pltpu.VMEM((T, MS), jnp.float32),
            pltpu.VMEM((T, MS), jnp.float32),
            pltpu.VMEM((T, 1), jnp.float32),
            pltpu.VMEM((T, E), jnp.float32),
            pltpu.VMEM((T, 2 * M), jnp.float32),
            pltpu.VMEM((T, M), jnp.float32),
        ],
        compiler_params=pltpu.CompilerParams(
            dimension_semantics=("arbitrary",)),
    )(flat, router_w, shared_eg_w, shared_gate_w, shared_inter_w,
      expert_gate_w, expert_out_w, shared_out_w)

    return (out_flat.reshape(B, S, H), logits)


# speedup vs baseline: 1.3154x; 1.0306x over previous
import jax
import jax.numpy as jnp
from jax.experimental import pallas as pl
from jax.experimental.pallas import tpu as pltpu

H = 2048
M = 1408
MS = 5632
E = 16
T = 32

P1C = 128
NP1 = H // P1C
EHC = 512
NEH = H // EHC
DCH = H // NEH
SKC = 512
NSK = MS // SKC

NE = E * NEH
NSTEPS = NE + NEH
J_ACT = 17
J_P2 = NSTEPS - NSK


def _body(flat_ref, rw_ref, eg_ref, sg_ref, si_ref, gate_ref, outw_ref,
          so_ref, out_ref, logits_ref, g_ref, x_ref, h_sh_ref, seg_ref,
          rout_ref, gu_ref, he_ref):
    j = pl.program_id(0)

    @pl.when(j == 0)
    def _router():
        flat = flat_ref[...]
        logits = jnp.dot(flat, rw_ref[...], preferred_element_type=jnp.float32)
        logits_ref[...] = logits
        m = jnp.max(logits, axis=-1, keepdims=True)
        ex = jnp.exp(logits - m)
        probs = ex / jnp.sum(ex, axis=-1, keepdims=True)
        lane = jax.lax.broadcasted_iota(jnp.int32, probs.shape, 1)
        p1 = jnp.max(probs, axis=-1, keepdims=True)
        i1 = jnp.min(jnp.where(probs == p1, lane, E), axis=-1, keepdims=True)
        is1 = lane == i1
        probs2 = jnp.where(is1, -1.0, probs)
        p2 = jnp.max(probs2, axis=-1, keepdims=True)
        i2 = jnp.min(jnp.where(probs2 == p2, lane, E), axis=-1, keepdims=True)
        is2 = lane == i2
        s = p1 + p2
        rout_ref[...] = (jnp.where(is1, p1 / s, 0.0)
                         + jnp.where(is2, p2 / s, 0.0))
        seg_ref[...] = jnp.dot(flat, eg_ref[...],
                               preferred_element_type=jnp.float32)

    @pl.when(j < NP1)
    def _phase1():
        fc = flat_ref[:, pl.ds(j * P1C, P1C)]
        gp = jnp.dot(fc, sg_ref[...], preferred_element_type=jnp.float32)
        xp = jnp.dot(fc, si_ref[...], preferred_element_type=jnp.float32)

        @pl.when(j == 0)
        def _reset():
            g_ref[...] = gp
            x_ref[...] = xp

        @pl.when(j != 0)
        def _accum():
            g_ref[...] += gp
            x_ref[...] += xp

    @pl.when(j == J_ACT)
    def _activate():
        g = g_ref[...]
        h_sh_ref[...] = x_ref[...] * (g * jax.nn.sigmoid(g)) * \
            jax.nn.sigmoid(seg_ref[...])

    @pl.when(j >= NEH)
    def _down():
        kd = j - NEH
        cd = kd % NEH
        contrib = jnp.dot(he_ref[...], outw_ref[0],
                          preferred_element_type=jnp.float32)

        @pl.when(kd < NEH)
        def _init():
            out_ref[:, pl.ds(cd * DCH, DCH)] = contrib

        @pl.when(kd >= NEH)
        def _add():
            out_ref[:, pl.ds(cd * DCH, DCH)] += contrib

    @pl.when(j < NE)
    def _expert():
        k = j % NEH
        fc = flat_ref[:, pl.ds(k * EHC, EHC)]
        part = jnp.dot(fc, gate_ref[0], preferred_element_type=jnp.float32)

        @pl.when(k == 0)
        def _reset():
            gu_ref[...] = part

        @pl.when(k != 0)
        def _accum():
            gu_ref[...] += part

        @pl.when(k == NEH - 1)
        def _act_e():
            e = j // NEH
            gu = gu_ref[...]
            g = gu[:, :M]
            u = gu[:, M:]
            lane = jax.lax.broadcasted_iota(jnp.int32, (T, E), 1)
            w = jnp.sum(jnp.where(lane == e, rout_ref[...], 0.0), axis=1,
                        keepdims=True)
            he_ref[...] = (g * jax.nn.sigmoid(g)) * u * w

    @pl.when(j >= J_P2)
    def _phase2():
        ks = j - J_P2
        hc = h_sh_ref[:, pl.ds(ks * SKC, SKC)]
        out_ref[...] += jnp.dot(hc, so_ref[...],
                                preferred_element_type=jnp.float32)


def kernel(hidden_states, router_w, expert_gate_w, expert_out_w,
           shared_gate_w, shared_inter_w, shared_out_w, shared_eg_w):
    B, S, _ = hidden_states.shape
    flat = hidden_states.reshape(-1, H)

    def _e_idx(j):
        ke = jnp.clip(j, 0, NE - 1)
        return (ke // NEH, ke % NEH, 0)

    def _d_idx(j):
        kd = jnp.clip(j - NEH, 0, NE - 1)
        return (kd // NEH, 0, kd % NEH)

    out_flat, logits = pl.pallas_call(
        _body,
        grid=(NSTEPS,),
        in_specs=[
            pl.BlockSpec((T, H), lambda j: (0, 0)),
            pl.BlockSpec((H, E), lambda j: (0, 0)),
            pl.BlockSpec((H, 1), lambda j: (0, 0)),
            pl.BlockSpec((P1C, MS), lambda j: (jnp.clip(j, 0, NP1 - 1), 0)),
            pl.BlockSpec((P1C, MS), lambda j: (jnp.clip(j, 0, NP1 - 1), 0)),
            pl.BlockSpec((1, EHC, 2 * M), _e_idx),
            pl.BlockSpec((1, M, DCH), _d_idx),
            pl.BlockSpec((SKC, H), lambda j: (jnp.clip(j - J_P2, 0, NSK - 1), 0)),
        ],
        out_specs=(
            pl.BlockSpec((T, H), lambda j: (0, 0)),
            pl.BlockSpec((T, E), lambda j: (0, 0)),
        ),
        out_shape=(
            jax.ShapeDtypeStruct((T, H), jnp.float32),
            jax.ShapeDtypeStruct((T, E), jnp.float32),
        ),
        scratch_shapes=[
            pltpu.VMEM((T, MS), jnp.float32),
            pltpu.VMEM((T, MS), jnp.float32),
            pltpu.VMEM((T, MS), jnp.float32),
            pltpu.VMEM((T, 1), jnp.float32),
            pltpu.VMEM((T, E), jnp.float32),
            pltpu.VMEM((T, 2 * M), jnp.float32),
            pltpu.VMEM((T, M), jnp.float32),
        ],
        compiler_params=pltpu.CompilerParams(
            dimension_semantics=("arbitrary",)),
    )(flat, router_w, shared_eg_w, shared_gate_w, shared_inter_w,
      expert_gate_w, expert_out_w, shared_out_w)

    return (out_flat.reshape(B, S, H), logits)
